# Initial kernel scaffold; baseline (speedup 1.0000x reference)
import jax
import jax.numpy as jnp
from jax.experimental import pallas as pl


def kernel(x, edge_index, edge_dist, weight, attention):
    raise NotImplementedError("write your pallas kernel here")


# speedup vs baseline: 4.7645x; 4.7645x over previous
import functools

import jax
import jax.numpy as jnp
from jax import lax
from jax.experimental import pallas as pl
from jax.experimental.pallas import tpu as pltpu
from jax.experimental.pallas import tpu_sc as plsc

_L = 16
_NS = 16
_NC = 2
_NW = _NC * _NS


def _cdiv(a, b):
    return (a + b - 1) // b


def _matvec_body(x_ref, w_ref, o_ref):
    w1 = jnp.sum(w_ref[0], axis=1, keepdims=True) * 2.0
    o_ref[...] = jnp.dot(x_ref[...], w1, preferred_element_type=jnp.float32)


@functools.lru_cache(maxsize=None)
def _make_sc_kernel(N, E):
    CG = _cdiv(N, _NS * 128) * 128
    NP = _NS * CG
    GR3 = CG // 128
    KH = _cdiv(_cdiv(E, _NS * 128), 8) * 8
    EP = _NS * 128 * KH
    E2 = E + N
    R4 = _cdiv(E2, _NW * 128)
    OP = _NW * 128 * R4
    C4 = R4 * 128
    GR = 9
    G4, REM4 = divmod(R4, GR)
    GH = 12
    G1, REM1 = divmod(KH, GH)
    NU = _cdiv(N, _L)

    mesh = plsc.VectorSubcoreMesh(core_axis_name="c", subcore_axis_name="s")

    def body(src2d, src1, dst1, ed, perm1, s_in, out,
             cnt_sh, g_sh, u_v, g_v, s_v, cnt_v, srcbuf, permbuf3,
             gath_s, gath_e, gbuf, permbuf4, srcg, dstg, resbuf, ones_v,
             zerobuf, sem):
        cid = lax.axis_index("c")
        tid = lax.axis_index("s")
        wid = cid * _NS + tid

        def fill16(i, _):
            ones_v[pl.ds(i * _L, _L)] = jnp.ones((_L,), jnp.float32)
            return _
        lax.fori_loop(0, 128 // _L, fill16, None)

        def zero16(i, _):
            zerobuf[pl.ds(i * _L, _L)] = jnp.zeros((_L,), jnp.float32)
            return _
        lax.fori_loop(0, CG // _L, zero16, None)
        pltpu.sync_copy(zerobuf, cnt_sh.at[pl.ds(tid * CG, CG)])
        plsc.subcore_barrier()

        pltpu.sync_copy(src2d.at[pl.ds(tid * KH, KH)], srcbuf)

        def hist_group(gi, _):
            descs = []
            for r in range(GH):
                descs.append(pltpu.async_copy(
                    ones_v, cnt_sh.at[srcbuf.at[gi * GH + r]], sem, add=True))
            for d in descs:
                d.wait()
            return _
        lax.fori_loop(0, G1, hist_group, None)
        descs = []
        for r in range(REM1):
            descs.append(pltpu.async_copy(
                ones_v, cnt_sh.at[srcbuf.at[G1 * GH + r]], sem, add=True))
        for d in descs:
            d.wait()
        plsc.subcore_barrier()

        pltpu.sync_copy(cnt_sh, cnt_v)
        pltpu.sync_copy(s_in, s_v)

        def u_step(i, _):
            sl = pl.ds(i * _L, _L)
            u_v[sl] = s_v[sl] / (cnt_v[sl] + 1.0)
            return _
        lax.fori_loop(0, NU, u_step, None)

        pltpu.sync_copy(perm1.at[pl.ds(tid * CG, CG)], permbuf3)
        descs = []
        for r in range(GR3):
            idx = permbuf3.at[pl.ds(r * 128, 128)]
            descs.append(pltpu.async_copy(
                src1.at[idx], gath_s.at[pl.ds(r * 128, 128)], sem))
            descs.append(pltpu.async_copy(
                ed.at[idx], gath_e.at[pl.ds(r * 128, 128)], sem))
        for d in descs:
            d.wait()

        def g_step(i, _):
            sl = pl.ds(i * _L, _L)
            uv = plsc.load_gather(u_v, [gath_s[sl]])
            gbuf[sl] = uv / gath_e[sl]
            return _
        lax.fori_loop(0, CG // _L, g_step, None)
        pltpu.sync_copy(gbuf, g_sh.at[pl.ds(tid * CG, CG)])
        plsc.subcore_barrier()
        pltpu.sync_copy(g_sh, g_v)

        pltpu.sync_copy(perm1.at[pl.ds(wid * C4, C4)], permbuf4)

        def do_rows(base_row, nrows):
            descs = []
            for r in range(nrows):
                start = pl.multiple_of((base_row + r) * 128, 128)
                idx = permbuf4.at[pl.ds(start, 128)]
                descs.append(pltpu.async_copy(src1.at[idx], srcg.at[r], sem))
                descs.append(pltpu.async_copy(dst1.at[idx], dstg.at[r], sem))
            for d in descs:
                d.wait()
            for r in range(nrows):
                for j in range(128 // _L):
                    sl = pl.ds(j * _L, _L)
                    gs = plsc.load_gather(g_v, [srcg.at[r][sl]])
                    gd = plsc.load_gather(g_v, [dstg.at[r][sl]])
                    resbuf[pl.ds((base_row + r) * 128 + j * _L, _L)] = gs - gd

        def p4_group(gi, _):
            do_rows(gi * GR, GR)
            return _
        lax.fori_loop(0, G4, p4_group, None)
        if REM4:
            do_rows(G4 * GR, REM4)
        pltpu.sync_copy(resbuf, out.at[pl.ds(wid * C4, C4)])

    return pl.kernel(
        body,
        out_type=jax.ShapeDtypeStruct((OP,), jnp.float32),
        mesh=mesh,
        scratch_types=[
            pltpu.VMEM_SHARED((NP,), jnp.float32),
            pltpu.VMEM_SHARED((NP,), jnp.float32),
            pltpu.VMEM((NP,), jnp.float32),
            pltpu.VMEM((NP,), jnp.float32),
            pltpu.VMEM((N,), jnp.float32),
            pltpu.VMEM((NP,), jnp.float32),
            pltpu.VMEM((KH, 128), jnp.int32),
            pltpu.VMEM((CG,), jnp.int32),
            pltpu.VMEM((CG,), jnp.int32),
            pltpu.VMEM((CG,), jnp.float32),
            pltpu.VMEM((CG,), jnp.float32),
            pltpu.VMEM((C4,), jnp.int32),
            pltpu.VMEM((GR, 128), jnp.int32),
            pltpu.VMEM((GR, 128), jnp.int32),
            pltpu.VMEM((C4,), jnp.float32),
            pltpu.VMEM((128,), jnp.float32),
            pltpu.VMEM((CG,), jnp.float32),
            pltpu.SemaphoreType.DMA,
        ],
        compiler_params=pltpu.CompilerParams(needs_layout_passes=False),
        name="gat2_sc",
    )


def kernel(x, edge_index, edge_dist, weight, attention):
    N, _ = x.shape
    E = edge_index.shape[0]
    h = weight.shape[2]

    src = edge_index[:, 0].astype(jnp.int32)
    dst = edge_index[:, 1].astype(jnp.int32)
    ed = edge_dist.astype(jnp.float32)

    perm = jnp.argsort(src * jnp.int32(N) + dst).astype(jnp.int32)

    CG = _cdiv(N, _NS * 128) * 128
    NP = _NS * CG
    KH = _cdiv(_cdiv(E, _NS * 128), 8) * 8
    EP = _NS * 128 * KH
    E2 = E + N
    R4 = _cdiv(E2, _NW * 128)
    OP = _NW * 128 * R4

    pad_node = jnp.int32(NP - 1)
    src1 = jnp.concatenate([src, jnp.full((EP - E,), pad_node, jnp.int32)])
    dst1 = jnp.concatenate([dst, jnp.full((EP - E,), pad_node, jnp.int32)])
    src2d = src1.reshape(EP // 128, 128)
    perm1 = jnp.concatenate(
        [perm, jnp.full((OP - E,), jnp.int32(E), jnp.int32)])

    s_col = pl.pallas_call(
        _matvec_body,
        out_shape=jax.ShapeDtypeStruct((N, 1), jnp.float32),
        name="gat2_matvec",
    )(x, weight)
    s_in = s_col[:, 0]

    out_pad = _make_sc_kernel(N, E)(src2d, src1, dst1, ed, perm1, s_in)
    return out_pad[:E2].reshape(-1, h)
